# baseline (speedup 1.0000x reference)
import jax
import jax.numpy as jnp
from jax import lax
from jax.experimental import pallas as pl

E = 8
D = 1024
R = 128
TILE = 1024


def _ffn_body(x_ref, c_ref, uall_ref, vall_ref, b_ref, o_ref):
    xt = x_ref[...]
    c = c_ref[...]
    xn = jnp.sum(xt * xt, axis=1, keepdims=True)
    dots = lax.dot_general(xt, c, (((1,), (1,)), ((), ())),
                           preferred_element_type=jnp.float32)
    cn = jnp.sum(c * c, axis=1)
    d2 = xn - 2.0 * dots + cn[None, :]
    bestv = d2[:, 0:1]
    bestid = jnp.zeros((xt.shape[0], 1), dtype=jnp.int32)
    for e in range(1, E):
        v = d2[:, e:e + 1]
        take = v < bestv
        bestid = jnp.where(take, e, bestid)
        bestv = jnp.where(take, v, bestv)
    h = lax.dot_general(xt, uall_ref[...], (((1,), (0,)), ((), ())),
                        preferred_element_type=jnp.float32)
    lane_eid = lax.broadcasted_iota(jnp.int32, (1, E * R), 1) // R
    hm = jnp.where(bestid == lane_eid, h, 0.0)
    out = lax.dot_general(hm, vall_ref[...], (((1,), (0,)), ((), ())),
                          preferred_element_type=jnp.float32)
    o_ref[...] = out + b_ref[...]


@jax.jit
def kernel(x, centroids, U, V, bias):
    orig_shape = x.shape
    xf = x.reshape(-1, x.shape[-1])
    n = xf.shape[0]
    grid = n // TILE
    U_all = U.transpose(1, 0, 2).reshape(D, E * R)
    V_all = V.reshape(E * R, D)
    out = pl.pallas_call(
        _ffn_body,
        grid=(grid,),
        in_specs=[
            pl.BlockSpec((TILE, D), lambda i: (i, 0)),
            pl.BlockSpec((E, D), lambda i: (0, 0)),
            pl.BlockSpec((D, E * R), lambda i: (0, 0)),
            pl.BlockSpec((E * R, D), lambda i: (0, 0)),
            pl.BlockSpec((1, D), lambda i: (0, 0)),
        ],
        out_specs=pl.BlockSpec((TILE, D), lambda i: (i, 0)),
        out_shape=jax.ShapeDtypeStruct((n, D), jnp.float32),
    )(xf, centroids, U_all, V_all, bias.reshape(1, D))
    return out.reshape(orig_shape)


# speedup vs baseline: 1.0240x; 1.0240x over previous
import jax
import jax.numpy as jnp
from jax import lax
from jax.experimental import pallas as pl

E = 8
D = 1024
R = 128
TILE = 512


def _ffn_body(x_ref, c_ref, uall_ref, vall_ref, b_ref, o_ref):
    xt = x_ref[...]
    c = c_ref[...]
    xn = jnp.sum(xt * xt, axis=1, keepdims=True)
    dots = lax.dot_general(xt, c, (((1,), (1,)), ((), ())),
                           preferred_element_type=jnp.float32)
    cn = jnp.sum(c * c, axis=1)
    d2 = xn - 2.0 * dots + cn[None, :]
    bestv = d2[:, 0:1]
    bestid = jnp.zeros((xt.shape[0], 1), dtype=jnp.int32)
    for e in range(1, E):
        v = d2[:, e:e + 1]
        take = v < bestv
        bestid = jnp.where(take, e, bestid)
        bestv = jnp.where(take, v, bestv)
    h = lax.dot_general(xt, uall_ref[...], (((1,), (0,)), ((), ())),
                        preferred_element_type=jnp.float32)
    lane_eid = lax.broadcasted_iota(jnp.int32, (1, E * R), 1) // R
    hm = jnp.where(bestid == lane_eid, h, 0.0)
    out = lax.dot_general(hm, vall_ref[...], (((1,), (0,)), ((), ())),
                          preferred_element_type=jnp.float32)
    o_ref[...] = out + b_ref[...]


@jax.jit
def kernel(x, centroids, U, V, bias):
    orig_shape = x.shape
    xf = x.reshape(-1, x.shape[-1])
    n = xf.shape[0]
    grid = n // TILE
    U_all = U.transpose(1, 0, 2).reshape(D, E * R)
    V_all = V.reshape(E * R, D)
    out = pl.pallas_call(
        _ffn_body,
        grid=(grid,),
        in_specs=[
            pl.BlockSpec((TILE, D), lambda i: (i, 0)),
            pl.BlockSpec((E, D), lambda i: (0, 0)),
            pl.BlockSpec((D, E * R), lambda i: (0, 0)),
            pl.BlockSpec((E * R, D), lambda i: (0, 0)),
            pl.BlockSpec((1, D), lambda i: (0, 0)),
        ],
        out_specs=pl.BlockSpec((TILE, D), lambda i: (i, 0)),
        out_shape=jax.ShapeDtypeStruct((n, D), jnp.float32),
    )(xf, centroids, U_all, V_all, bias.reshape(1, D))
    return out.reshape(orig_shape)
